# baseline (speedup 1.0000x reference)
import functools

import jax
import jax.numpy as jnp
from jax import lax
from jax.experimental import pallas as pl
from jax.experimental.pallas import tpu as pltpu
from jax.experimental.pallas import tpu_sc as plsc

D_MODEL = 1024
VOCAB = 100000


_VB = 512


def _transpose_body(w_ref, out_ref):
    out_ref[...] = w_ref[...].T


def _transpose_table(w):
    nblocks = pl.cdiv(VOCAB, _VB)
    return pl.pallas_call(
        _transpose_body,
        grid=(nblocks,),
        in_specs=[pl.BlockSpec((D_MODEL, _VB), lambda i: (0, i))],
        out_specs=pl.BlockSpec((_VB, D_MODEL), lambda i: (i, 0)),
        out_shape=jax.ShapeDtypeStruct((VOCAB, D_MODEL), w.dtype),
    )(w)


_NC = 2
_NS = 16
_NW = _NC * _NS
_B = 4 * 4096
_BPW = _B // _NW
_C = 32
_NCHUNK = _BPW // _C


def _gather_body(table_hbm, idx_hbm, out_hbm, idx_v, rows0, rows1,
                 gsem0, gsem1, wsem0, wsem1):
    wid = lax.axis_index("s") * _NC + lax.axis_index("c")
    base = wid * _BPW
    pltpu.sync_copy(idx_hbm.at[pl.ds(base, _BPW)], idx_v)

    bufs = (rows0, rows1)
    gsems = (gsem0, gsem1)
    wsems = (wsem0, wsem1)

    pltpu.async_copy(table_hbm.at[idx_v.at[pl.ds(0, _C)]], rows0, gsem0)
    for c in range(_NCHUNK):
        k = c % 2
        nk = (c + 1) % 2
        if c + 1 < _NCHUNK:
            if c >= 1:
                pltpu.make_async_copy(
                    bufs[nk], out_hbm.at[pl.ds(base + (c - 1) * _C, _C)],
                    wsems[nk]).wait()
            pltpu.async_copy(
                table_hbm.at[idx_v.at[pl.ds((c + 1) * _C, _C)]],
                bufs[nk], gsems[nk])
        pltpu.make_async_copy(
            table_hbm.at[idx_v.at[pl.ds(c * _C, _C)]], bufs[k],
            gsems[k]).wait()
        pltpu.async_copy(bufs[k], out_hbm.at[pl.ds(base + c * _C, _C)],
                         wsems[k])
    pltpu.make_async_copy(
        bufs[(_NCHUNK - 2) % 2],
        out_hbm.at[pl.ds(base + (_NCHUNK - 2) * _C, _C)],
        wsems[(_NCHUNK - 2) % 2]).wait()
    pltpu.make_async_copy(
        bufs[(_NCHUNK - 1) % 2],
        out_hbm.at[pl.ds(base + (_NCHUNK - 1) * _C, _C)],
        wsems[(_NCHUNK - 1) % 2]).wait()


@functools.lru_cache(maxsize=None)
def _sc_gather_fn():
    return pl.kernel(
        _gather_body,
        out_type=jax.ShapeDtypeStruct((_B, D_MODEL), jnp.float32),
        mesh=plsc.VectorSubcoreMesh(
            core_axis_name="c", subcore_axis_name="s",
            num_cores=_NC, num_subcores=_NS),
        scratch_types=[
            pltpu.VMEM((_BPW,), jnp.int32),
            pltpu.VMEM((_C, D_MODEL), jnp.float32),
            pltpu.VMEM((_C, D_MODEL), jnp.float32),
            pltpu.SemaphoreType.DMA,
            pltpu.SemaphoreType.DMA,
            pltpu.SemaphoreType.DMA,
            pltpu.SemaphoreType.DMA,
        ],
    )


def kernel(tokens, W_E):
    table = _transpose_table(W_E)
    idx = tokens.reshape(-1).astype(jnp.int32)
    out = _sc_gather_fn()(table, idx)
    return out.reshape(tokens.shape[0], tokens.shape[1], D_MODEL)


# speedup vs baseline: 10.7605x; 10.7605x over previous
import functools

import jax
import jax.numpy as jnp
from jax import lax
from jax.experimental import pallas as pl
from jax.experimental.pallas import tpu as pltpu
from jax.experimental.pallas import tpu_sc as plsc

D_MODEL = 1024
VOCAB = 100000


_NC = 2
_NS = 16
_NW = _NC * _NS
_B = 4 * 4096
_BPW = _B // _NW
_C = 32
_NCHUNK = _BPW // _C


def _gather_body(table_hbm, idx_hbm, out_hbm, idx_v, rows0, rows1,
                 gsem0, gsem1, wsem0, wsem1):
    wid = lax.axis_index("s") * _NC + lax.axis_index("c")
    base = wid * _BPW
    pltpu.sync_copy(idx_hbm.at[pl.ds(base, _BPW)], idx_v)

    bufs = (rows0, rows1)
    gsems = (gsem0, gsem1)
    wsems = (wsem0, wsem1)

    pltpu.async_copy(table_hbm.at[idx_v.at[pl.ds(0, _C)]], rows0, gsem0)
    for c in range(_NCHUNK):
        k = c % 2
        nk = (c + 1) % 2
        if c + 1 < _NCHUNK:
            if c >= 1:
                pltpu.make_async_copy(
                    bufs[nk], out_hbm.at[pl.ds(base + (c - 1) * _C, _C)],
                    wsems[nk]).wait()
            pltpu.async_copy(
                table_hbm.at[idx_v.at[pl.ds((c + 1) * _C, _C)]],
                bufs[nk], gsems[nk])
        pltpu.make_async_copy(
            table_hbm.at[idx_v.at[pl.ds(c * _C, _C)]], bufs[k],
            gsems[k]).wait()
        pltpu.async_copy(bufs[k], out_hbm.at[pl.ds(base + c * _C, _C)],
                         wsems[k])
    pltpu.make_async_copy(
        bufs[(_NCHUNK - 2) % 2],
        out_hbm.at[pl.ds(base + (_NCHUNK - 2) * _C, _C)],
        wsems[(_NCHUNK - 2) % 2]).wait()
    pltpu.make_async_copy(
        bufs[(_NCHUNK - 1) % 2],
        out_hbm.at[pl.ds(base + (_NCHUNK - 1) * _C, _C)],
        wsems[(_NCHUNK - 1) % 2]).wait()


@functools.lru_cache(maxsize=None)
def _sc_gather_fn():
    return pl.kernel(
        _gather_body,
        out_type=jax.ShapeDtypeStruct((_B, D_MODEL), jnp.float32),
        mesh=plsc.VectorSubcoreMesh(
            core_axis_name="c", subcore_axis_name="s",
            num_cores=_NC, num_subcores=_NS),
        scratch_types=[
            pltpu.VMEM((_BPW,), jnp.int32),
            pltpu.VMEM((_C, D_MODEL), jnp.float32),
            pltpu.VMEM((_C, D_MODEL), jnp.float32),
            pltpu.SemaphoreType.DMA,
            pltpu.SemaphoreType.DMA,
            pltpu.SemaphoreType.DMA,
            pltpu.SemaphoreType.DMA,
        ],
    )


def kernel(tokens, W_E):
    table = jnp.transpose(W_E)
    idx = tokens.reshape(-1).astype(jnp.int32)
    out = _sc_gather_fn()(table, idx)
    return out.reshape(tokens.shape[0], tokens.shape[1], D_MODEL)


# speedup vs baseline: 10.9385x; 1.0165x over previous
import functools

import jax
import jax.numpy as jnp
from jax import lax
from jax.experimental import pallas as pl
from jax.experimental.pallas import tpu as pltpu
from jax.experimental.pallas import tpu_sc as plsc

D_MODEL = 1024
VOCAB = 100000


_NC = 2
_NS = 16
_NW = _NC * _NS
_B = 4 * 4096
_BPW = _B // _NW
_C = 32
_NCHUNK = _BPW // _C


_NBUF = 3


def _gather_body(table_hbm, idx_hbm, out_hbm, idx_v, rows, gsems, wsems):
    wid = lax.axis_index("s") * _NC + lax.axis_index("c")
    base = wid * _BPW
    pltpu.sync_copy(idx_hbm.at[pl.ds(base, _BPW)], idx_v)

    def gather(c):
        b = c % _NBUF
        pltpu.async_copy(
            table_hbm.at[idx_v.at[pl.ds(c * _C, _C)]], rows[b], gsems[b])

    def wait_gather(c):
        b = c % _NBUF
        pltpu.make_async_copy(
            table_hbm.at[idx_v.at[pl.ds(c * _C, _C)]], rows[b],
            gsems[b]).wait()

    def write(c):
        b = c % _NBUF
        pltpu.async_copy(rows[b], out_hbm.at[pl.ds(base + c * _C, _C)],
                         wsems[b])

    def wait_write(c):
        b = c % _NBUF
        pltpu.make_async_copy(
            rows[b], out_hbm.at[pl.ds(base + c * _C, _C)], wsems[b]).wait()

    for c in range(_NBUF):
        gather(c)
    for c in range(_NCHUNK):
        wait_gather(c)
        write(c)
        g = c + _NBUF
        if g < _NCHUNK:
            wait_write(g - _NBUF)
            gather(g)
    for c in range(_NCHUNK - _NBUF, _NCHUNK):
        wait_write(c)


@functools.lru_cache(maxsize=None)
def _sc_gather_fn():
    return pl.kernel(
        _gather_body,
        out_type=jax.ShapeDtypeStruct((_B, D_MODEL), jnp.float32),
        mesh=plsc.VectorSubcoreMesh(
            core_axis_name="c", subcore_axis_name="s",
            num_cores=_NC, num_subcores=_NS),
        scratch_types=[
            pltpu.VMEM((_BPW,), jnp.int32),
            [pltpu.VMEM((_C, D_MODEL), jnp.float32) for _ in range(_NBUF)],
            [pltpu.SemaphoreType.DMA for _ in range(_NBUF)],
            [pltpu.SemaphoreType.DMA for _ in range(_NBUF)],
        ],
    )


def kernel(tokens, W_E):
    table = jnp.transpose(W_E)
    idx = tokens.reshape(-1).astype(jnp.int32)
    out = _sc_gather_fn()(table, idx)
    return out.reshape(tokens.shape[0], tokens.shape[1], D_MODEL)


# speedup vs baseline: 10.9947x; 1.0051x over previous
import functools

import jax
import jax.numpy as jnp
from jax import lax
from jax.experimental import pallas as pl
from jax.experimental.pallas import tpu as pltpu
from jax.experimental.pallas import tpu_sc as plsc

D_MODEL = 1024
VOCAB = 100000


_NC = 2
_NS = 16
_NW = _NC * _NS
_B = 4 * 4096
_BPW = _B // _NW
_C = 16
_NCHUNK = _BPW // _C


_NBUF = 6


def _gather_body(table_hbm, idx_hbm, out_hbm, idx_v, rows, gsems, wsems):
    wid = lax.axis_index("s") * _NC + lax.axis_index("c")
    base = wid * _BPW
    pltpu.sync_copy(idx_hbm.at[pl.ds(base, _BPW)], idx_v)

    def gather(c):
        b = c % _NBUF
        pltpu.async_copy(
            table_hbm.at[idx_v.at[pl.ds(c * _C, _C)]], rows[b], gsems[b])

    def wait_gather(c):
        b = c % _NBUF
        pltpu.make_async_copy(
            table_hbm.at[idx_v.at[pl.ds(c * _C, _C)]], rows[b],
            gsems[b]).wait()

    def write(c):
        b = c % _NBUF
        pltpu.async_copy(rows[b], out_hbm.at[pl.ds(base + c * _C, _C)],
                         wsems[b])

    def wait_write(c):
        b = c % _NBUF
        pltpu.make_async_copy(
            rows[b], out_hbm.at[pl.ds(base + c * _C, _C)], wsems[b]).wait()

    for c in range(_NBUF - 1):
        gather(c)
    for c in range(_NCHUNK):
        g = c + _NBUF - 1
        if g < _NCHUNK:
            if g >= _NBUF:
                wait_write(g - _NBUF)
            gather(g)
        wait_gather(c)
        write(c)
    for c in range(_NCHUNK - _NBUF, _NCHUNK):
        wait_write(c)


@functools.lru_cache(maxsize=None)
def _sc_gather_fn():
    return pl.kernel(
        _gather_body,
        out_type=jax.ShapeDtypeStruct((_B, D_MODEL), jnp.float32),
        mesh=plsc.VectorSubcoreMesh(
            core_axis_name="c", subcore_axis_name="s",
            num_cores=_NC, num_subcores=_NS),
        scratch_types=[
            pltpu.VMEM((_BPW,), jnp.int32),
            [pltpu.VMEM((_C, D_MODEL), jnp.float32) for _ in range(_NBUF)],
            [pltpu.SemaphoreType.DMA for _ in range(_NBUF)],
            [pltpu.SemaphoreType.DMA for _ in range(_NBUF)],
        ],
    )


def kernel(tokens, W_E):
    table = jnp.transpose(W_E)
    idx = tokens.reshape(-1).astype(jnp.int32)
    out = _sc_gather_fn()(table, idx)
    return out.reshape(tokens.shape[0], tokens.shape[1], D_MODEL)


# speedup vs baseline: 11.1693x; 1.0159x over previous
import functools

import jax
import jax.numpy as jnp
from jax import lax
from jax.experimental import pallas as pl
from jax.experimental.pallas import tpu as pltpu
from jax.experimental.pallas import tpu_sc as plsc

D_MODEL = 1024
VOCAB = 100000


_NC = 2
_NS = 16
_NW = _NC * _NS
_B = 4 * 4096
_BPW = _B // _NW
_C = 16
_NCHUNK = _BPW // _C


_NBUF = 4


def _gather_body(table_hbm, idx_hbm, out_hbm, idx_v, rows, gsems, wsems):
    wid = lax.axis_index("s") * _NC + lax.axis_index("c")
    base = wid * _BPW
    pltpu.sync_copy(idx_hbm.at[pl.ds(base, _BPW)], idx_v)

    def gather(c, b):
        pltpu.async_copy(
            table_hbm.at[idx_v.at[pl.ds(c * _C, _C)]], rows[b], gsems[b])

    def wait_gather(c, b):
        pltpu.make_async_copy(
            table_hbm.at[idx_v.at[pl.ds(c * _C, _C)]], rows[b],
            gsems[b]).wait()

    def write(c, b):
        pltpu.async_copy(rows[b], out_hbm.at[pl.ds(base + c * _C, _C)],
                         wsems[b])

    def wait_write(c, b):
        pltpu.make_async_copy(
            rows[b], out_hbm.at[pl.ds(base + c * _C, _C)], wsems[b]).wait()

    for c in range(_NBUF):
        gather(c, c % _NBUF)

    @pl.loop(0, _NCHUNK - _NBUF, step=_NBUF)
    def _steady(c0):
        for j in range(_NBUF):
            c = c0 + j
            wait_gather(c, j)
            write(c, j)
            wait_write(c, j)
            gather(c + _NBUF, j)

    for c in range(_NCHUNK - _NBUF, _NCHUNK):
        wait_gather(c, c % _NBUF)
        write(c, c % _NBUF)
    for c in range(_NCHUNK - _NBUF, _NCHUNK):
        wait_write(c, c % _NBUF)


@functools.lru_cache(maxsize=None)
def _sc_gather_fn():
    return pl.kernel(
        _gather_body,
        out_type=jax.ShapeDtypeStruct((_B, D_MODEL), jnp.float32),
        mesh=plsc.VectorSubcoreMesh(
            core_axis_name="c", subcore_axis_name="s",
            num_cores=_NC, num_subcores=_NS),
        scratch_types=[
            pltpu.VMEM((_BPW,), jnp.int32),
            [pltpu.VMEM((_C, D_MODEL), jnp.float32) for _ in range(_NBUF)],
            [pltpu.SemaphoreType.DMA for _ in range(_NBUF)],
            [pltpu.SemaphoreType.DMA for _ in range(_NBUF)],
        ],
    )


def kernel(tokens, W_E):
    table = jnp.transpose(W_E)
    idx = tokens.reshape(-1).astype(jnp.int32)
    out = _sc_gather_fn()(table, idx)
    return out.reshape(tokens.shape[0], tokens.shape[1], D_MODEL)
